# Initial kernel scaffold; baseline (speedup 1.0000x reference)
import jax
import jax.numpy as jnp
from jax.experimental import pallas as pl


def kernel(x, edge_index, batch, W_in, b_in, conv_Wl, conv_bl, conv_Wr, ln_g, ln_b, W_out, b_out):
    raise NotImplementedError("write your pallas kernel here")


# speedup vs baseline: 4.6763x; 4.6763x over previous
import functools

import jax
import jax.numpy as jnp
from jax import lax
from jax.experimental import pallas as pl
from jax.experimental.pallas import tpu as pltpu
from jax.experimental.pallas import tpu_sc as plsc

NC = 2
NS = 16
NW = NC * NS


@functools.partial(jax.jit, static_argnames=("n_pad", "h", "e"))
def _sc_segsum(g, src, dst, zeros_tile, n_pad, h, e):
  epw = e // NW
  ch = 80
  nchunk = epw // ch
  rpt = n_pad // NS
  n = n_pad

  mesh = plsc.VectorSubcoreMesh(
      core_axis_name="c", subcore_axis_name="s", num_cores=NC,
      num_subcores=NS)

  @functools.partial(
      pl.kernel,
      mesh=mesh,
      out_type=jax.ShapeDtypeStruct((NC, n, h), jnp.float32),
      scratch_types=[
          pltpu.VMEM((ch,), jnp.int32),
          pltpu.VMEM((ch,), jnp.int32),
          pltpu.VMEM((ch, h), jnp.float32),
          pltpu.VMEM_SHARED((n, h), jnp.float32),
          pltpu.SemaphoreType.DMA,
      ],
  )
  def seg_kernel(g_hbm, src_hbm, dst_hbm, z_hbm, out_hbm,
                 srcb, dstb, rows, acc, sem):
    c = lax.axis_index("c")
    s = lax.axis_index("s")
    wid = s * NC + c
    r0 = s * rpt

    pltpu.sync_copy(z_hbm, acc.at[pl.ds(r0, rpt)])
    plsc.subcore_barrier()

    def step(j, carry):
      off = wid * epw + j * ch
      pltpu.sync_copy(src_hbm.at[pl.ds(off, ch)], srcb)
      pltpu.sync_copy(dst_hbm.at[pl.ds(off, ch)], dstb)
      pltpu.async_copy(g_hbm.at[srcb], rows, sem).wait()
      pltpu.sync_copy(rows, acc.at[dstb], add=True)
      return carry

    lax.fori_loop(0, nchunk, step, 0)
    plsc.subcore_barrier()
    pltpu.sync_copy(acc.at[pl.ds(r0, rpt)], out_hbm.at[c, pl.ds(r0, rpt)])

  return seg_kernel(g, src, dst, zeros_tile)


@functools.partial(jax.jit, static_argnames=("n_pad", "e", "w"))
def _sc_counts(dst, ones_rows, zeros_tile, n_pad, e, w):
  epw = e // NW
  ch = 80
  nchunk = epw // ch
  rpt = n_pad // NS
  n = n_pad

  mesh = plsc.VectorSubcoreMesh(
      core_axis_name="c", subcore_axis_name="s", num_cores=NC,
      num_subcores=NS)

  @functools.partial(
      pl.kernel,
      mesh=mesh,
      out_type=jax.ShapeDtypeStruct((NC, n, w), jnp.float32),
      scratch_types=[
          pltpu.VMEM((ch,), jnp.int32),
          pltpu.VMEM((ch, w), jnp.float32),
          pltpu.VMEM_SHARED((n, w), jnp.float32),
          pltpu.SemaphoreType.DMA,
      ],
  )
  def cnt_kernel(dst_hbm, ones_hbm, z_hbm, out_hbm, dstb, ones_v, acc, sem):
    c = lax.axis_index("c")
    s = lax.axis_index("s")
    wid = s * NC + c
    r0 = s * rpt

    pltpu.sync_copy(z_hbm, acc.at[pl.ds(r0, rpt)])
    pltpu.sync_copy(ones_hbm, ones_v)
    plsc.subcore_barrier()

    def step(j, carry):
      off = wid * epw + j * ch
      pltpu.sync_copy(dst_hbm.at[pl.ds(off, ch)], dstb)
      pltpu.sync_copy(ones_v, acc.at[dstb], add=True)
      return carry

    lax.fori_loop(0, nchunk, step, 0)
    plsc.subcore_barrier()
    pltpu.sync_copy(acc.at[pl.ds(r0, rpt)], out_hbm.at[c, pl.ds(r0, rpt)])

  return cnt_kernel(dst, ones_rows, zeros_tile)


def _dot(a, b):
  return jnp.dot(a, b, preferred_element_type=jnp.float32)


def _tc_entry(x, w_in, b_in, wl0):
  n, _ = x.shape
  h_dim = w_in.shape[1]

  def body(x_ref, win_ref, bin_ref, wl0_ref, h_ref, g_ref):
    hh = jnp.maximum(_dot(x_ref[...], win_ref[...]) + bin_ref[...], 0.0)
    h_ref[...] = hh
    g_ref[...] = _dot(hh, wl0_ref[...])

  return pl.pallas_call(
      body,
      out_shape=(
          jax.ShapeDtypeStruct((n, h_dim), jnp.float32),
          jax.ShapeDtypeStruct((n, h_dim), jnp.float32),
      ),
  )(x, w_in, b_in, wl0)


def _tc_layer(h, part, cntp, wr, bl, g_ln, b_ln, wl_next, w_out_i, y_in):
  n, h_dim = h.shape
  c_dim = w_out_i.shape[1]
  have_next = wl_next is not None

  def body(*refs):
    if have_next:
      (h_ref, p_ref, cnt_ref, wr_ref, bl_ref, gln_ref, bln_ref, wn_ref,
       wo_ref, y_ref, ho_ref, go_ref, yo_ref) = refs
    else:
      (h_ref, p_ref, cnt_ref, wr_ref, bl_ref, gln_ref, bln_ref,
       wo_ref, y_ref, ho_ref, yo_ref) = refs
    cnt = cnt_ref[0, :n] + cnt_ref[1, :n]
    rc = 1.0 / jnp.maximum(cnt[:, 0:1], 1.0)
    hh = h_ref[...]
    p = p_ref[0, :n] + p_ref[1, :n]
    t = p * rc + bl_ref[...] + _dot(hh, wr_ref[...])
    mu = jnp.mean(t, axis=-1, keepdims=True)
    var = jnp.mean((t - mu) * (t - mu), axis=-1, keepdims=True)
    t = (t - mu) * lax.rsqrt(var + 1e-5) * gln_ref[...] + bln_ref[...]
    hn = jnp.maximum(t, 0.0) + hh
    ho_ref[...] = hn
    if have_next:
      go_ref[...] = _dot(hn, wn_ref[...])
    yo_ref[...] = y_ref[...] + _dot(hn, wo_ref[...])

  out_shape = [jax.ShapeDtypeStruct((n, h_dim), jnp.float32)]
  if have_next:
    out_shape.append(jax.ShapeDtypeStruct((n, h_dim), jnp.float32))
  out_shape.append(jax.ShapeDtypeStruct((n, c_dim), jnp.float32))

  args = [h, part, cntp, wr, bl, g_ln, b_ln]
  if have_next:
    args.append(wl_next)
  args += [w_out_i, y_in]

  return pl.pallas_call(body, out_shape=tuple(out_shape))(*args)


def kernel(x, edge_index, batch, W_in, b_in, conv_Wl, conv_bl, conv_Wr,
           ln_g, ln_b, W_out, b_out):
  n, _ = x.shape
  h_dim = W_in.shape[1]
  num_layers = conv_Wl.shape[0]
  c_dim = W_out.shape[1]
  e = edge_index.shape[1]

  src = edge_index[0]
  dst = edge_index[1]
  n_pad = -(-n // (8 * NS)) * (8 * NS)
  rpt = n_pad // NS
  zeros_h = jnp.zeros((rpt, h_dim), jnp.float32)
  ones_rows = jnp.ones((80, h_dim), jnp.float32)

  cntp = _sc_counts(dst, ones_rows, zeros_h, n_pad=n_pad, e=e, w=h_dim)
  cnt_small = cntp[:, :, :8]

  h, g = _tc_entry(x, W_in, b_in.reshape(1, h_dim), conv_Wl[0])

  y = jnp.broadcast_to(b_out.reshape(1, c_dim), (n, c_dim))
  for i in range(num_layers):
    part = _sc_segsum(g, src, dst, zeros_h, n_pad=n_pad, h=h_dim, e=e)
    wl_next = conv_Wl[i + 1] if i + 1 < num_layers else None
    outs = _tc_layer(
        h, part, cnt_small, conv_Wr[i], conv_bl[i].reshape(1, h_dim),
        ln_g[i].reshape(1, h_dim), ln_b[i].reshape(1, h_dim), wl_next,
        W_out[i * h_dim:(i + 1) * h_dim], y)
    if wl_next is not None:
      h, g, y = outs
    else:
      h, y = outs
  return y


# speedup vs baseline: 6.9390x; 1.4839x over previous
import functools

import jax
import jax.numpy as jnp
from jax import lax
from jax.experimental import pallas as pl
from jax.experimental.pallas import tpu as pltpu
from jax.experimental.pallas import tpu_sc as plsc

NC = 2
NS = 16
NW = NC * NS


@functools.partial(jax.jit, static_argnames=("n_pad", "h", "e"))
def _sc_segsum(g, src, dst, zeros_tile, n_pad, h, e):
  epw = e // NW
  ch = 80
  nchunk = epw // ch
  rpt = n_pad // NS
  n = n_pad

  mesh = plsc.VectorSubcoreMesh(
      core_axis_name="c", subcore_axis_name="s", num_cores=NC,
      num_subcores=NS)

  assert nchunk % 2 == 1
  npair = nchunk // 2

  @functools.partial(
      pl.kernel,
      mesh=mesh,
      out_type=jax.ShapeDtypeStruct((NC, n, h), jnp.float32),
      scratch_types=[
          pltpu.VMEM((ch,), jnp.int32),
          pltpu.VMEM((ch,), jnp.int32),
          pltpu.VMEM((ch,), jnp.int32),
          pltpu.VMEM((ch,), jnp.int32),
          pltpu.VMEM((ch, h), jnp.float32),
          pltpu.VMEM((ch, h), jnp.float32),
          pltpu.VMEM_SHARED((n, h), jnp.float32),
          pltpu.SemaphoreType.DMA,
          pltpu.SemaphoreType.DMA,
      ],
  )
  def seg_kernel(g_hbm, src_hbm, dst_hbm, z_hbm, out_hbm,
                 srcb0, srcb1, dstb0, dstb1, rows0, rows1, acc, sem0, sem1):
    c = lax.axis_index("c")
    s = lax.axis_index("s")
    wid = s * NC + c
    r0 = s * rpt
    base = wid * epw

    pltpu.sync_copy(z_hbm, acc.at[pl.ds(r0, rpt)])
    plsc.subcore_barrier()

    pltpu.sync_copy(src_hbm.at[pl.ds(base, ch)], srcb0)
    pltpu.sync_copy(dst_hbm.at[pl.ds(base, ch)], dstb0)
    pltpu.async_copy(g_hbm.at[srcb0], rows0, sem0)

    def step(j, carry):
      o1 = base + (2 * j + 1) * ch
      o2 = base + (2 * j + 2) * ch
      pltpu.sync_copy(src_hbm.at[pl.ds(o1, ch)], srcb1)
      pltpu.sync_copy(dst_hbm.at[pl.ds(o1, ch)], dstb1)
      pltpu.async_copy(g_hbm.at[srcb1], rows1, sem1)
      pltpu.make_async_copy(g_hbm.at[srcb0], rows0, sem0).wait()
      pltpu.sync_copy(rows0, acc.at[dstb0], add=True)
      pltpu.sync_copy(src_hbm.at[pl.ds(o2, ch)], srcb0)
      pltpu.sync_copy(dst_hbm.at[pl.ds(o2, ch)], dstb0)
      pltpu.async_copy(g_hbm.at[srcb0], rows0, sem0)
      pltpu.make_async_copy(g_hbm.at[srcb1], rows1, sem1).wait()
      pltpu.sync_copy(rows1, acc.at[dstb1], add=True)
      return carry

    lax.fori_loop(0, npair, step, 0)
    pltpu.make_async_copy(g_hbm.at[srcb0], rows0, sem0).wait()
    pltpu.sync_copy(rows0, acc.at[dstb0], add=True)
    plsc.subcore_barrier()
    pltpu.sync_copy(acc.at[pl.ds(r0, rpt)], out_hbm.at[c, pl.ds(r0, rpt)])

  return seg_kernel(g, src, dst, zeros_tile)


@functools.partial(jax.jit, static_argnames=("n_pad", "e", "w"))
def _sc_counts(dst, ones_rows, zeros_tile, n_pad, e, w):
  epw = e // NW
  ch = 80
  nchunk = epw // ch
  rpt = n_pad // NS
  n = n_pad

  mesh = plsc.VectorSubcoreMesh(
      core_axis_name="c", subcore_axis_name="s", num_cores=NC,
      num_subcores=NS)

  @functools.partial(
      pl.kernel,
      mesh=mesh,
      out_type=jax.ShapeDtypeStruct((NC, n, w), jnp.float32),
      scratch_types=[
          pltpu.VMEM((ch,), jnp.int32),
          pltpu.VMEM((ch, w), jnp.float32),
          pltpu.VMEM_SHARED((n, w), jnp.float32),
          pltpu.SemaphoreType.DMA,
      ],
  )
  def cnt_kernel(dst_hbm, ones_hbm, z_hbm, out_hbm, dstb, ones_v, acc, sem):
    c = lax.axis_index("c")
    s = lax.axis_index("s")
    wid = s * NC + c
    r0 = s * rpt

    pltpu.sync_copy(z_hbm, acc.at[pl.ds(r0, rpt)])
    pltpu.sync_copy(ones_hbm, ones_v)
    plsc.subcore_barrier()

    def step(j, carry):
      off = wid * epw + j * ch
      pltpu.sync_copy(dst_hbm.at[pl.ds(off, ch)], dstb)
      pltpu.sync_copy(ones_v, acc.at[dstb], add=True)
      return carry

    lax.fori_loop(0, nchunk, step, 0)
    plsc.subcore_barrier()
    pltpu.sync_copy(acc.at[pl.ds(r0, rpt)], out_hbm.at[c, pl.ds(r0, rpt)])

  return cnt_kernel(dst, ones_rows, zeros_tile)


def _dot(a, b):
  return jnp.dot(a, b, preferred_element_type=jnp.float32)


def _tc_entry(x, w_in, b_in, wl0):
  n, _ = x.shape
  h_dim = w_in.shape[1]

  def body(x_ref, win_ref, bin_ref, wl0_ref, h_ref, g_ref):
    hh = jnp.maximum(_dot(x_ref[...], win_ref[...]) + bin_ref[...], 0.0)
    h_ref[...] = hh
    g_ref[...] = _dot(hh, wl0_ref[...])

  return pl.pallas_call(
      body,
      out_shape=(
          jax.ShapeDtypeStruct((n, h_dim), jnp.float32),
          jax.ShapeDtypeStruct((n, h_dim), jnp.float32),
      ),
  )(x, w_in, b_in, wl0)


def _tc_layer(h, part, cntp, wr, bl, g_ln, b_ln, wl_next, w_out_i, y_in):
  n, h_dim = h.shape
  c_dim = w_out_i.shape[1]
  have_next = wl_next is not None

  def body(*refs):
    if have_next:
      (h_ref, p_ref, cnt_ref, wr_ref, bl_ref, gln_ref, bln_ref, wn_ref,
       wo_ref, y_ref, ho_ref, go_ref, yo_ref) = refs
    else:
      (h_ref, p_ref, cnt_ref, wr_ref, bl_ref, gln_ref, bln_ref,
       wo_ref, y_ref, ho_ref, yo_ref) = refs
    cnt = cnt_ref[0, :n] + cnt_ref[1, :n]
    rc = 1.0 / jnp.maximum(cnt[:, 0:1], 1.0)
    hh = h_ref[...]
    p = p_ref[0, :n] + p_ref[1, :n]
    t = p * rc + bl_ref[...] + _dot(hh, wr_ref[...])
    mu = jnp.mean(t, axis=-1, keepdims=True)
    var = jnp.mean((t - mu) * (t - mu), axis=-1, keepdims=True)
    t = (t - mu) * lax.rsqrt(var + 1e-5) * gln_ref[...] + bln_ref[...]
    hn = jnp.maximum(t, 0.0) + hh
    ho_ref[...] = hn
    if have_next:
      go_ref[...] = _dot(hn, wn_ref[...])
    yo_ref[...] = y_ref[...] + _dot(hn, wo_ref[...])

  out_shape = [jax.ShapeDtypeStruct((n, h_dim), jnp.float32)]
  if have_next:
    out_shape.append(jax.ShapeDtypeStruct((n, h_dim), jnp.float32))
  out_shape.append(jax.ShapeDtypeStruct((n, c_dim), jnp.float32))

  args = [h, part, cntp, wr, bl, g_ln, b_ln]
  if have_next:
    args.append(wl_next)
  args += [w_out_i, y_in]

  return pl.pallas_call(body, out_shape=tuple(out_shape))(*args)


def kernel(x, edge_index, batch, W_in, b_in, conv_Wl, conv_bl, conv_Wr,
           ln_g, ln_b, W_out, b_out):
  n, _ = x.shape
  h_dim = W_in.shape[1]
  num_layers = conv_Wl.shape[0]
  c_dim = W_out.shape[1]
  e = edge_index.shape[1]

  src = edge_index[0]
  dst = edge_index[1]
  n_pad = -(-n // (8 * NS)) * (8 * NS)
  rpt = n_pad // NS
  zeros_h = jnp.zeros((rpt, h_dim), jnp.float32)
  ones_rows = jnp.ones((80, h_dim), jnp.float32)

  cntp = _sc_counts(dst, ones_rows, zeros_h, n_pad=n_pad, e=e, w=h_dim)
  cnt_small = cntp[:, :, :8]

  h, g = _tc_entry(x, W_in, b_in.reshape(1, h_dim), conv_Wl[0])

  y = jnp.broadcast_to(b_out.reshape(1, c_dim), (n, c_dim))
  for i in range(num_layers):
    part = _sc_segsum(g, src, dst, zeros_h, n_pad=n_pad, h=h_dim, e=e)
    wl_next = conv_Wl[i + 1] if i + 1 < num_layers else None
    outs = _tc_layer(
        h, part, cnt_small, conv_Wr[i], conv_bl[i].reshape(1, h_dim),
        ln_g[i].reshape(1, h_dim), ln_b[i].reshape(1, h_dim), wl_next,
        W_out[i * h_dim:(i + 1) * h_dim], y)
    if wl_next is not None:
      h, g, y = outs
    else:
      h, y = outs
  return y
